# Initial kernel scaffold; baseline (speedup 1.0000x reference)
import jax
import jax.numpy as jnp
from jax.experimental import pallas as pl


def kernel(x, edge_index, W_mlp, b_mlp, W_conv, b_conv, W_ih, W_hh, b_ih, b_hh):
    raise NotImplementedError("write your pallas kernel here")


# speedup vs baseline: 6.2364x; 6.2364x over previous
import functools

import jax
import jax.numpy as jnp
from jax import lax
from jax.experimental import pallas as pl
from jax.experimental.pallas import tpu as pltpu
from jax.experimental.pallas import tpu_sc as plsc

_N = 10000
_DI = 128
_DO = 64
_STEPS = 2

_NC = 2
_NS = 16
_NW = _NC * _NS
_CHUNK = 128
_TILE_ROWS = 640
_ACC_ROWS = _NS * _TILE_ROWS
_DUMMY = _N


def _mlp_conv_body(x_ref, wm_ref, bm_ref, wc_ref, bc_ref, h_ref, lin_ref):
    o = jnp.dot(x_ref[...], wm_ref[...], preferred_element_type=jnp.float32)
    o = o + bm_ref[...]
    h_ref[...] = o
    lin_ref[...] = (
        jnp.dot(o, wc_ref[...], preferred_element_type=jnp.float32) + bc_ref[...]
    )


def _gru_body(p0_ref, p1_ref, x_ref, h_ref, wm_ref, wx_ref, wh_ref,
              bi_ref, bh_ref, wc_ref, bc_ref, hn_ref, lin_ref):
    m = p0_ref[...] + p1_ref[...]
    gi = (
        jnp.dot(m, wm_ref[...], preferred_element_type=jnp.float32)
        + jnp.dot(x_ref[...], wx_ref[...], preferred_element_type=jnp.float32)
        + bi_ref[...]
    )
    gh = jnp.dot(h_ref[...], wh_ref[...], preferred_element_type=jnp.float32)
    gh = gh + bh_ref[...]
    h = h_ref[...]
    r = jax.nn.sigmoid(gi[:, 0:_DO] + gh[:, 0:_DO])
    z = jax.nn.sigmoid(gi[:, _DO:2 * _DO] + gh[:, _DO:2 * _DO])
    n = jnp.tanh(gi[:, 2 * _DO:3 * _DO] + r * gh[:, 2 * _DO:3 * _DO])
    hn = (1.0 - z) * n + z * h
    hn_ref[...] = hn
    lin_ref[...] = (
        jnp.dot(hn, wc_ref[...], preferred_element_type=jnp.float32) + bc_ref[...]
    )


_G = 5
_BLK = _N // _G


def _tc_mlp_conv(x, W_mlp, b_mlp, W_conv, b_conv):
    return pl.pallas_call(
        _mlp_conv_body,
        grid=(_G,),
        in_specs=[
            pl.BlockSpec((_BLK, _DI), lambda i: (i, 0)),
            pl.BlockSpec((_DI, _DO), lambda i: (0, 0)),
            pl.BlockSpec((1, _DO), lambda i: (0, 0)),
            pl.BlockSpec((_DO, _DO), lambda i: (0, 0)),
            pl.BlockSpec((1, _DO), lambda i: (0, 0)),
        ],
        out_specs=[
            pl.BlockSpec((_BLK, _DO), lambda i: (i, 0)),
            pl.BlockSpec((_BLK, _DO), lambda i: (i, 0)),
        ],
        out_shape=[
            jax.ShapeDtypeStruct((_N, _DO), jnp.float32),
            jax.ShapeDtypeStruct((_N, _DO), jnp.float32),
        ],
    )(x, W_mlp, b_mlp, W_conv, b_conv)


def _tc_gru(p0, p1, x, h, wm, wx, wh, bi, bh, wc, bc):
    return pl.pallas_call(
        _gru_body,
        grid=(_G,),
        in_specs=[
            pl.BlockSpec((_BLK, _DO), lambda i: (i, 0)),
            pl.BlockSpec((_BLK, _DO), lambda i: (i, 0)),
            pl.BlockSpec((_BLK, _DI), lambda i: (i, 0)),
            pl.BlockSpec((_BLK, _DO), lambda i: (i, 0)),
            pl.BlockSpec((_DO, 3 * _DO), lambda i: (0, 0)),
            pl.BlockSpec((_DI, 3 * _DO), lambda i: (0, 0)),
            pl.BlockSpec((_DO, 3 * _DO), lambda i: (0, 0)),
            pl.BlockSpec((1, 3 * _DO), lambda i: (0, 0)),
            pl.BlockSpec((1, 3 * _DO), lambda i: (0, 0)),
            pl.BlockSpec((_DO, _DO), lambda i: (0, 0)),
            pl.BlockSpec((1, _DO), lambda i: (0, 0)),
        ],
        out_specs=[
            pl.BlockSpec((_BLK, _DO), lambda i: (i, 0)),
            pl.BlockSpec((_BLK, _DO), lambda i: (i, 0)),
        ],
        out_shape=[
            jax.ShapeDtypeStruct((_N, _DO), jnp.float32),
            jax.ShapeDtypeStruct((_N, _DO), jnp.float32),
        ],
    )(p0, p1, x, h, wm, wx, wh, bi, bh, wc, bc)


def _make_sc_scatter(chunks):
    mesh = plsc.VectorSubcoreMesh(core_axis_name="c", subcore_axis_name="s")

    @functools.partial(
        pl.kernel,
        mesh=mesh,
        out_type=jax.ShapeDtypeStruct((_NC, _ACC_ROWS, _DO), jnp.float32),
        compiler_params=pltpu.CompilerParams(use_tc_tiling_on_sc=False),
        scratch_types=[
            pltpu.VMEM((chunks, _CHUNK), jnp.int32),
            pltpu.VMEM((chunks, _CHUNK), jnp.int32),
            pltpu.VMEM((_CHUNK, _DO), jnp.float32),
            pltpu.VMEM((_TILE_ROWS, _DO), jnp.float32),
            pltpu.VMEM_SHARED((_ACC_ROWS, _DO), jnp.float32),
            pltpu.SemaphoreType.DMA,
        ],
    )
    def sc_scatter(lin_hbm, src_hbm, dst_hbm, zeros_hbm, out_hbm,
                   src_v, dst_v, rows_v, stage_v, acc_sh, sem):
        c = lax.axis_index("c")
        s = lax.axis_index("s")
        wid = s * _NC + c
        row0 = s * _TILE_ROWS
        pltpu.sync_copy(zeros_hbm, stage_v)
        pltpu.sync_copy(stage_v, acc_sh.at[pl.ds(row0, _TILE_ROWS)])
        pltpu.sync_copy(src_hbm.at[wid], src_v)
        pltpu.sync_copy(dst_hbm.at[wid], dst_v)
        plsc.subcore_barrier()

        def chunk(j, carry):
            pltpu.async_copy(lin_hbm.at[src_v.at[j]], rows_v, sem).wait()
            pltpu.sync_copy(rows_v, acc_sh.at[dst_v.at[j]], add=True)
            return carry

        lax.fori_loop(0, chunks, chunk, 0)
        plsc.subcore_barrier()
        pltpu.sync_copy(acc_sh.at[pl.ds(row0, _TILE_ROWS)], stage_v)
        pltpu.sync_copy(stage_v, out_hbm.at[c, pl.ds(row0, _TILE_ROWS)])

    return sc_scatter


def kernel(x, edge_index, W_mlp, b_mlp, W_conv, b_conv, W_ih, W_hh, b_ih, b_hh):
    n_edges = edge_index.shape[1]
    e_per_w = -(-n_edges // (_NW * _CHUNK)) * _CHUNK
    chunks = e_per_w // _CHUNK
    e_pad = _NW * e_per_w

    src = edge_index[0].astype(jnp.int32)
    dst = edge_index[1].astype(jnp.int32)
    pad = e_pad - n_edges
    src3 = jnp.concatenate([src, jnp.zeros((pad,), jnp.int32)]).reshape(
        _NW, chunks, _CHUNK)
    dst3 = jnp.concatenate([dst, jnp.full((pad,), _DUMMY, jnp.int32)]).reshape(
        _NW, chunks, _CHUNK)
    zeros_tile = jnp.zeros((_TILE_ROWS, _DO), jnp.float32)

    bm = b_mlp.reshape(1, _DO)
    bc = b_conv.reshape(1, _DO)
    wm = W_ih[:, :_DO].T
    wx = W_ih[:, _DO:].T
    wh = W_hh.T
    bi = b_ih.reshape(1, 3 * _DO)
    bh = b_hh.reshape(1, 3 * _DO)

    sc_scatter = _make_sc_scatter(chunks)

    h, lin = _tc_mlp_conv(x, W_mlp, bm, W_conv, bc)
    for _ in range(_STEPS):
        parts = sc_scatter(lin, src3, dst3, zeros_tile)
        p0 = parts[0, :_N]
        p1 = parts[1, :_N]
        h, lin = _tc_gru(p0, p1, x, h, wm, wx, wh, bi, bh, W_conv, bc)
    return h


# speedup vs baseline: 9.3513x; 1.4995x over previous
import functools

import jax
import jax.numpy as jnp
from jax import lax
from jax.experimental import pallas as pl
from jax.experimental.pallas import tpu as pltpu
from jax.experimental.pallas import tpu_sc as plsc

_N = 10000
_DI = 128
_DO = 64
_STEPS = 2

_NC = 2
_NS = 16
_NW = _NC * _NS
_CHUNK = 128
_TILE_ROWS = 640
_ACC_ROWS = _NS * _TILE_ROWS
_DUMMY = _N


def _mlp_conv_body(x_ref, wm_ref, bm_ref, wc_ref, bc_ref, h_ref, lin_ref):
    o = jnp.dot(x_ref[...], wm_ref[...], preferred_element_type=jnp.float32)
    o = o + bm_ref[...]
    h_ref[...] = o
    lin_ref[...] = (
        jnp.dot(o, wc_ref[...], preferred_element_type=jnp.float32) + bc_ref[...]
    )


def _gru_body(p0_ref, p1_ref, x_ref, h_ref, wm_ref, wx_ref, wh_ref,
              bi_ref, bh_ref, wc_ref, bc_ref, hn_ref, lin_ref):
    m = p0_ref[...] + p1_ref[...]
    gi = (
        jnp.dot(m, wm_ref[...], preferred_element_type=jnp.float32)
        + jnp.dot(x_ref[...], wx_ref[...], preferred_element_type=jnp.float32)
        + bi_ref[...]
    )
    gh = jnp.dot(h_ref[...], wh_ref[...], preferred_element_type=jnp.float32)
    gh = gh + bh_ref[...]
    h = h_ref[...]
    r = jax.nn.sigmoid(gi[:, 0:_DO] + gh[:, 0:_DO])
    z = jax.nn.sigmoid(gi[:, _DO:2 * _DO] + gh[:, _DO:2 * _DO])
    n = jnp.tanh(gi[:, 2 * _DO:3 * _DO] + r * gh[:, 2 * _DO:3 * _DO])
    hn = (1.0 - z) * n + z * h
    hn_ref[...] = hn
    lin_ref[...] = (
        jnp.dot(hn, wc_ref[...], preferred_element_type=jnp.float32) + bc_ref[...]
    )


_G = 5
_BLK = _N // _G


def _tc_mlp_conv(x, W_mlp, b_mlp, W_conv, b_conv):
    return pl.pallas_call(
        _mlp_conv_body,
        grid=(_G,),
        in_specs=[
            pl.BlockSpec((_BLK, _DI), lambda i: (i, 0)),
            pl.BlockSpec((_DI, _DO), lambda i: (0, 0)),
            pl.BlockSpec((1, _DO), lambda i: (0, 0)),
            pl.BlockSpec((_DO, _DO), lambda i: (0, 0)),
            pl.BlockSpec((1, _DO), lambda i: (0, 0)),
        ],
        out_specs=[
            pl.BlockSpec((_BLK, _DO), lambda i: (i, 0)),
            pl.BlockSpec((_BLK, _DO), lambda i: (i, 0)),
        ],
        out_shape=[
            jax.ShapeDtypeStruct((_N, _DO), jnp.float32),
            jax.ShapeDtypeStruct((_N, _DO), jnp.float32),
        ],
    )(x, W_mlp, b_mlp, W_conv, b_conv)


def _tc_gru(p0, p1, x, h, wm, wx, wh, bi, bh, wc, bc):
    return pl.pallas_call(
        _gru_body,
        grid=(_G,),
        in_specs=[
            pl.BlockSpec((_BLK, _DO), lambda i: (i, 0)),
            pl.BlockSpec((_BLK, _DO), lambda i: (i, 0)),
            pl.BlockSpec((_BLK, _DI), lambda i: (i, 0)),
            pl.BlockSpec((_BLK, _DO), lambda i: (i, 0)),
            pl.BlockSpec((_DO, 3 * _DO), lambda i: (0, 0)),
            pl.BlockSpec((_DI, 3 * _DO), lambda i: (0, 0)),
            pl.BlockSpec((_DO, 3 * _DO), lambda i: (0, 0)),
            pl.BlockSpec((1, 3 * _DO), lambda i: (0, 0)),
            pl.BlockSpec((1, 3 * _DO), lambda i: (0, 0)),
            pl.BlockSpec((_DO, _DO), lambda i: (0, 0)),
            pl.BlockSpec((1, _DO), lambda i: (0, 0)),
        ],
        out_specs=[
            pl.BlockSpec((_BLK, _DO), lambda i: (i, 0)),
            pl.BlockSpec((_BLK, _DO), lambda i: (i, 0)),
        ],
        out_shape=[
            jax.ShapeDtypeStruct((_N, _DO), jnp.float32),
            jax.ShapeDtypeStruct((_N, _DO), jnp.float32),
        ],
    )(p0, p1, x, h, wm, wx, wh, bi, bh, wc, bc)


_NBUF = 4


def _make_sc_scatter(chunks):
    mesh = plsc.VectorSubcoreMesh(core_axis_name="c", subcore_axis_name="s")
    groups = chunks // _NBUF

    @functools.partial(
        pl.kernel,
        mesh=mesh,
        out_type=jax.ShapeDtypeStruct((_NC, _ACC_ROWS, _DO), jnp.float32),
        compiler_params=pltpu.CompilerParams(use_tc_tiling_on_sc=False),
        scratch_types=[
            pltpu.VMEM((chunks, _CHUNK), jnp.int32),
            pltpu.VMEM((chunks, _CHUNK), jnp.int32),
            pltpu.VMEM((_CHUNK, _DO), jnp.float32),
            pltpu.VMEM_SHARED((_ACC_ROWS, _DO), jnp.float32),
            pltpu.VMEM_SHARED((_ACC_ROWS, _DO), jnp.float32),
            pltpu.SemaphoreType.DMA,
        ],
    )
    def sc_scatter(lin_hbm, src_hbm, dst_hbm, zeros_hbm, out_hbm,
                   src_v, dst_v, rows_v, lin_sh, acc_sh, sem):
        c = lax.axis_index("c")
        s = lax.axis_index("s")
        wid = s * _NC + c
        row0 = s * _TILE_ROWS
        stripe = pl.ds(row0, _TILE_ROWS)
        pltpu.sync_copy(lin_hbm.at[stripe], lin_sh.at[stripe])
        pltpu.sync_copy(zeros_hbm, acc_sh.at[stripe])
        pltpu.sync_copy(src_hbm.at[wid], src_v)
        pltpu.sync_copy(dst_hbm.at[wid], dst_v)
        plsc.subcore_barrier()

        def chunk(j, carry):
            pltpu.async_copy(lin_sh.at[src_v.at[j]], rows_v, sem).wait()
            pltpu.sync_copy(rows_v, acc_sh.at[dst_v.at[j]], add=True)
            return carry

        lax.fori_loop(0, chunks, chunk, 0)
        plsc.subcore_barrier()
        pltpu.sync_copy(acc_sh.at[stripe], out_hbm.at[c, stripe])

    return sc_scatter


def kernel(x, edge_index, W_mlp, b_mlp, W_conv, b_conv, W_ih, W_hh, b_ih, b_hh):
    n_edges = edge_index.shape[1]
    e_per_w = -(-n_edges // (_NW * _CHUNK)) * _CHUNK
    chunks = e_per_w // _CHUNK
    e_pad = _NW * e_per_w

    src = edge_index[0].astype(jnp.int32)
    dst = edge_index[1].astype(jnp.int32)
    pad = e_pad - n_edges
    src3 = jnp.concatenate([src, jnp.zeros((pad,), jnp.int32)]).reshape(
        _NW, chunks, _CHUNK)
    dst3 = jnp.concatenate([dst, jnp.full((pad,), _DUMMY, jnp.int32)]).reshape(
        _NW, chunks, _CHUNK)
    zeros_tile = jnp.zeros((_TILE_ROWS, _DO), jnp.float32)
    lin_pad = jnp.zeros((_ACC_ROWS - _N, _DO), jnp.float32)

    bm = b_mlp.reshape(1, _DO)
    bc = b_conv.reshape(1, _DO)
    wm = W_ih[:, :_DO].T
    wx = W_ih[:, _DO:].T
    wh = W_hh.T
    bi = b_ih.reshape(1, 3 * _DO)
    bh = b_hh.reshape(1, 3 * _DO)

    sc_scatter = _make_sc_scatter(chunks)

    h, lin = _tc_mlp_conv(x, W_mlp, bm, W_conv, bc)
    for _ in range(_STEPS):
        lin_full = jnp.concatenate([lin, lin_pad], axis=0)
        parts = sc_scatter(lin_full, src3, dst3, zeros_tile)
        p0 = parts[0, :_N]
        p1 = parts[1, :_N]
        h, lin = _tc_gru(p0, p1, x, h, wm, wx, wh, bi, bh, W_conv, bc)
    return h


# speedup vs baseline: 9.7544x; 1.0431x over previous
import functools

import jax
import jax.numpy as jnp
from jax import lax
from jax.experimental import pallas as pl
from jax.experimental.pallas import tpu as pltpu
from jax.experimental.pallas import tpu_sc as plsc

_N = 10000
_DI = 128
_DO = 64
_STEPS = 2

_NC = 2
_NS = 16
_NW = _NC * _NS
_CHUNK = 128
_TILE_ROWS = 640
_ACC_ROWS = _NS * _TILE_ROWS
_DUMMY = _N


def _mlp_conv_body(x_ref, wm_ref, bm_ref, wc_ref, bc_ref, h_ref, lin_ref):
    o = jnp.dot(x_ref[...], wm_ref[...], preferred_element_type=jnp.float32)
    o = o + bm_ref[...]
    h_ref[...] = o
    lin_ref[...] = (
        jnp.dot(o, wc_ref[...], preferred_element_type=jnp.float32) + bc_ref[...]
    )


def _gru_body(p0_ref, p1_ref, x_ref, h_ref, wm_ref, wx_ref, wh_ref,
              bi_ref, bh_ref, wc_ref, bc_ref, hn_ref, lin_ref):
    m = p0_ref[0] + p1_ref[0]
    gi = (
        jnp.dot(m, wm_ref[...], preferred_element_type=jnp.float32)
        + jnp.dot(x_ref[...], wx_ref[...], preferred_element_type=jnp.float32)
        + bi_ref[...]
    )
    gh = jnp.dot(h_ref[...], wh_ref[...], preferred_element_type=jnp.float32)
    gh = gh + bh_ref[...]
    h = h_ref[...]
    r = jax.nn.sigmoid(gi[:, 0:_DO] + gh[:, 0:_DO])
    z = jax.nn.sigmoid(gi[:, _DO:2 * _DO] + gh[:, _DO:2 * _DO])
    n = jnp.tanh(gi[:, 2 * _DO:3 * _DO] + r * gh[:, 2 * _DO:3 * _DO])
    hn = (1.0 - z) * n + z * h
    hn_ref[...] = hn
    lin_ref[...] = (
        jnp.dot(hn, wc_ref[...], preferred_element_type=jnp.float32) + bc_ref[...]
    )


_G = 5
_BLK = _N // _G


def _tc_mlp_conv(x, W_mlp, b_mlp, W_conv, b_conv):
    return pl.pallas_call(
        _mlp_conv_body,
        grid=(_G,),
        in_specs=[
            pl.BlockSpec((_BLK, _DI), lambda i: (i, 0)),
            pl.BlockSpec((_DI, _DO), lambda i: (0, 0)),
            pl.BlockSpec((1, _DO), lambda i: (0, 0)),
            pl.BlockSpec((_DO, _DO), lambda i: (0, 0)),
            pl.BlockSpec((1, _DO), lambda i: (0, 0)),
        ],
        out_specs=[
            pl.BlockSpec((_BLK, _DO), lambda i: (i, 0)),
            pl.BlockSpec((_BLK, _DO), lambda i: (i, 0)),
        ],
        out_shape=[
            jax.ShapeDtypeStruct((_N, _DO), jnp.float32),
            jax.ShapeDtypeStruct((_ACC_ROWS, _DO), jnp.float32),
        ],
    )(x, W_mlp, b_mlp, W_conv, b_conv)


def _tc_gru(parts, x, h, wm, wx, wh, bi, bh, wc, bc):
    return pl.pallas_call(
        _gru_body,
        grid=(_G,),
        in_specs=[
            pl.BlockSpec((1, _BLK, _DO), lambda i: (0, i, 0)),
            pl.BlockSpec((1, _BLK, _DO), lambda i: (1, i, 0)),
            pl.BlockSpec((_BLK, _DI), lambda i: (i, 0)),
            pl.BlockSpec((_BLK, _DO), lambda i: (i, 0)),
            pl.BlockSpec((_DO, 3 * _DO), lambda i: (0, 0)),
            pl.BlockSpec((_DI, 3 * _DO), lambda i: (0, 0)),
            pl.BlockSpec((_DO, 3 * _DO), lambda i: (0, 0)),
            pl.BlockSpec((1, 3 * _DO), lambda i: (0, 0)),
            pl.BlockSpec((1, 3 * _DO), lambda i: (0, 0)),
            pl.BlockSpec((_DO, _DO), lambda i: (0, 0)),
            pl.BlockSpec((1, _DO), lambda i: (0, 0)),
        ],
        out_specs=[
            pl.BlockSpec((_BLK, _DO), lambda i: (i, 0)),
            pl.BlockSpec((_BLK, _DO), lambda i: (i, 0)),
        ],
        out_shape=[
            jax.ShapeDtypeStruct((_N, _DO), jnp.float32),
            jax.ShapeDtypeStruct((_ACC_ROWS, _DO), jnp.float32),
        ],
    )(parts, parts, x, h, wm, wx, wh, bi, bh, wc, bc)


_NBUF = 4


def _make_sc_scatter(chunks):
    mesh = plsc.VectorSubcoreMesh(core_axis_name="c", subcore_axis_name="s")
    groups = chunks // _NBUF

    @functools.partial(
        pl.kernel,
        mesh=mesh,
        out_type=jax.ShapeDtypeStruct((_NC, _ACC_ROWS, _DO), jnp.float32),
        compiler_params=pltpu.CompilerParams(use_tc_tiling_on_sc=False),
        scratch_types=[
            pltpu.VMEM((chunks, _CHUNK), jnp.int32),
            pltpu.VMEM((chunks, _CHUNK), jnp.int32),
            pltpu.VMEM((_CHUNK, _DO), jnp.float32),
            pltpu.VMEM_SHARED((_ACC_ROWS, _DO), jnp.float32),
            pltpu.VMEM_SHARED((_ACC_ROWS, _DO), jnp.float32),
            pltpu.SemaphoreType.DMA,
        ],
    )
    def sc_scatter(lin_hbm, src_hbm, dst_hbm, zeros_hbm, out_hbm,
                   src_v, dst_v, rows_v, lin_sh, acc_sh, sem):
        c = lax.axis_index("c")
        s = lax.axis_index("s")
        wid = s * _NC + c
        row0 = s * _TILE_ROWS
        stripe = pl.ds(row0, _TILE_ROWS)
        pltpu.sync_copy(lin_hbm.at[stripe], lin_sh.at[stripe])
        pltpu.sync_copy(zeros_hbm, acc_sh.at[stripe])
        pltpu.sync_copy(src_hbm.at[wid], src_v)
        pltpu.sync_copy(dst_hbm.at[wid], dst_v)
        plsc.subcore_barrier()

        def chunk(j, carry):
            pltpu.async_copy(lin_sh.at[src_v.at[j]], rows_v, sem).wait()
            pltpu.sync_copy(rows_v, acc_sh.at[dst_v.at[j]], add=True)
            return carry

        lax.fori_loop(0, chunks, chunk, 0)
        plsc.subcore_barrier()
        pltpu.sync_copy(acc_sh.at[stripe], out_hbm.at[c, stripe])

    return sc_scatter


def kernel(x, edge_index, W_mlp, b_mlp, W_conv, b_conv, W_ih, W_hh, b_ih, b_hh):
    n_edges = edge_index.shape[1]
    e_per_w = -(-n_edges // (_NW * _CHUNK)) * _CHUNK
    chunks = e_per_w // _CHUNK
    e_pad = _NW * e_per_w

    src = edge_index[0].astype(jnp.int32)
    dst = edge_index[1].astype(jnp.int32)
    pad = e_pad - n_edges
    src3 = jnp.concatenate([src, jnp.zeros((pad,), jnp.int32)]).reshape(
        _NW, chunks, _CHUNK)
    dst3 = jnp.concatenate([dst, jnp.full((pad,), _DUMMY, jnp.int32)]).reshape(
        _NW, chunks, _CHUNK)
    zeros_tile = jnp.zeros((_TILE_ROWS, _DO), jnp.float32)

    bm = b_mlp.reshape(1, _DO)
    bc = b_conv.reshape(1, _DO)
    wm = W_ih[:, :_DO].T
    wx = W_ih[:, _DO:].T
    wh = W_hh.T
    bi = b_ih.reshape(1, 3 * _DO)
    bh = b_hh.reshape(1, 3 * _DO)

    sc_scatter = _make_sc_scatter(chunks)

    h, lin = _tc_mlp_conv(x, W_mlp, bm, W_conv, bc)
    for _ in range(_STEPS):
        parts = sc_scatter(lin, src3, dst3, zeros_tile)
        h, lin = _tc_gru(parts, x, h, wm, wx, wh, bi, bh, W_conv, bc)
    return h


# speedup vs baseline: 9.8942x; 1.0143x over previous
import functools

import jax
import jax.numpy as jnp
from jax import lax
from jax.experimental import pallas as pl
from jax.experimental.pallas import tpu as pltpu
from jax.experimental.pallas import tpu_sc as plsc

_N = 10000
_DI = 128
_DO = 64
_STEPS = 2

_NC = 2
_NS = 16
_NW = _NC * _NS
_CHUNK = 128
_TILE_ROWS = 640
_ACC_ROWS = _NS * _TILE_ROWS
_DUMMY = _N


def _mlp_conv_body(x_ref, wm_ref, bm_ref, wc_ref, bc_ref, h_ref, lin_ref):
    o = jnp.dot(x_ref[...], wm_ref[...], preferred_element_type=jnp.float32)
    o = o + bm_ref[...]
    h_ref[...] = o
    lin_ref[...] = (
        jnp.dot(o, wc_ref[...], preferred_element_type=jnp.float32) + bc_ref[...]
    )


def _dot_t(a, w):
    return lax.dot_general(a, w, (((1,), (1,)), ((), ())),
                           preferred_element_type=jnp.float32)


def _gru_body(p0_ref, p1_ref, x_ref, h_ref, wih_ref, whh_ref,
              bi_ref, bh_ref, wc_ref, bc_ref, hn_ref, lin_ref):
    m = p0_ref[...] + p1_ref[...]
    wih = wih_ref[...]
    gi = (
        _dot_t(m, wih[:, :_DO])
        + _dot_t(x_ref[...], wih[:, _DO:])
        + bi_ref[...]
    )
    gh = _dot_t(h_ref[...], whh_ref[...])
    gh = gh + bh_ref[...]
    h = h_ref[...]
    r = jax.nn.sigmoid(gi[:, 0:_DO] + gh[:, 0:_DO])
    z = jax.nn.sigmoid(gi[:, _DO:2 * _DO] + gh[:, _DO:2 * _DO])
    n = jnp.tanh(gi[:, 2 * _DO:3 * _DO] + r * gh[:, 2 * _DO:3 * _DO])
    hn = (1.0 - z) * n + z * h
    hn_ref[...] = hn
    lin_ref[...] = (
        jnp.dot(hn, wc_ref[...], preferred_element_type=jnp.float32) + bc_ref[...]
    )


_G = 5
_BLK = _N // _G


def _tc_mlp_conv(x, W_mlp, b_mlp, W_conv, b_conv):
    return pl.pallas_call(
        _mlp_conv_body,
        grid=(_G,),
        in_specs=[
            pl.BlockSpec((_BLK, _DI), lambda i: (i, 0)),
            pl.BlockSpec((_DI, _DO), lambda i: (0, 0)),
            pl.BlockSpec((1, _DO), lambda i: (0, 0)),
            pl.BlockSpec((_DO, _DO), lambda i: (0, 0)),
            pl.BlockSpec((1, _DO), lambda i: (0, 0)),
        ],
        out_specs=[
            pl.BlockSpec((_BLK, _DO), lambda i: (i, 0)),
            pl.BlockSpec((_BLK, _DO), lambda i: (i, 0)),
        ],
        out_shape=[
            jax.ShapeDtypeStruct((_N, _DO), jnp.float32),
            jax.ShapeDtypeStruct((_ACC_ROWS, _DO), jnp.float32),
        ],
    )(x, W_mlp, b_mlp, W_conv, b_conv)


def _tc_gru(p0, p1, x, h, wih, whh, bi, bh, wc, bc):
    return pl.pallas_call(
        _gru_body,
        grid=(_G,),
        in_specs=[
            pl.BlockSpec((_BLK, _DO), lambda i: (i, 0)),
            pl.BlockSpec((_BLK, _DO), lambda i: (i, 0)),
            pl.BlockSpec((_BLK, _DI), lambda i: (i, 0)),
            pl.BlockSpec((_BLK, _DO), lambda i: (i, 0)),
            pl.BlockSpec((3 * _DO, _DO + _DI), lambda i: (0, 0)),
            pl.BlockSpec((3 * _DO, _DO), lambda i: (0, 0)),
            pl.BlockSpec((1, 3 * _DO), lambda i: (0, 0)),
            pl.BlockSpec((1, 3 * _DO), lambda i: (0, 0)),
            pl.BlockSpec((_DO, _DO), lambda i: (0, 0)),
            pl.BlockSpec((1, _DO), lambda i: (0, 0)),
        ],
        out_specs=[
            pl.BlockSpec((_BLK, _DO), lambda i: (i, 0)),
            pl.BlockSpec((_BLK, _DO), lambda i: (i, 0)),
        ],
        out_shape=[
            jax.ShapeDtypeStruct((_N, _DO), jnp.float32),
            jax.ShapeDtypeStruct((_ACC_ROWS, _DO), jnp.float32),
        ],
    )(p0, p1, x, h, wih, whh, bi, bh, wc, bc)


_NBUF = 4


def _make_sc_scatter(chunks):
    mesh = plsc.VectorSubcoreMesh(core_axis_name="c", subcore_axis_name="s")
    groups = chunks // _NBUF

    @functools.partial(
        pl.kernel,
        mesh=mesh,
        out_type=[
            jax.ShapeDtypeStruct((_ACC_ROWS, _DO), jnp.float32),
            jax.ShapeDtypeStruct((_ACC_ROWS, _DO), jnp.float32),
        ],
        compiler_params=pltpu.CompilerParams(use_tc_tiling_on_sc=False),
        scratch_types=[
            pltpu.VMEM((chunks, _CHUNK), jnp.int32),
            pltpu.VMEM((chunks, _CHUNK), jnp.int32),
            pltpu.VMEM((_CHUNK, _DO), jnp.float32),
            pltpu.VMEM_SHARED((_ACC_ROWS, _DO), jnp.float32),
            pltpu.VMEM_SHARED((_ACC_ROWS, _DO), jnp.float32),
            pltpu.SemaphoreType.DMA,
        ],
    )
    def sc_scatter(lin_hbm, src_hbm, dst_hbm, zeros_hbm, out0_hbm, out1_hbm,
                   src_v, dst_v, rows_v, lin_sh, acc_sh, sem):
        c = lax.axis_index("c")
        s = lax.axis_index("s")
        wid = s * _NC + c
        row0 = s * _TILE_ROWS
        stripe = pl.ds(row0, _TILE_ROWS)
        pltpu.sync_copy(lin_hbm.at[stripe], lin_sh.at[stripe])
        pltpu.sync_copy(zeros_hbm, acc_sh.at[stripe])
        pltpu.sync_copy(src_hbm.at[wid], src_v)
        pltpu.sync_copy(dst_hbm.at[wid], dst_v)
        plsc.subcore_barrier()

        def chunk(j, carry):
            pltpu.async_copy(lin_sh.at[src_v.at[j]], rows_v, sem).wait()
            pltpu.sync_copy(rows_v, acc_sh.at[dst_v.at[j]], add=True)
            return carry

        lax.fori_loop(0, chunks, chunk, 0)
        plsc.subcore_barrier()

        @pl.when(c == 0)
        def _():
            pltpu.sync_copy(acc_sh.at[stripe], out0_hbm.at[stripe])

        @pl.when(c == 1)
        def _():
            pltpu.sync_copy(acc_sh.at[stripe], out1_hbm.at[stripe])

    return sc_scatter


def kernel(x, edge_index, W_mlp, b_mlp, W_conv, b_conv, W_ih, W_hh, b_ih, b_hh):
    n_edges = edge_index.shape[1]
    e_per_w = -(-n_edges // (_NW * _CHUNK)) * _CHUNK
    chunks = e_per_w // _CHUNK
    e_pad = _NW * e_per_w

    src = edge_index[0].astype(jnp.int32)
    dst = edge_index[1].astype(jnp.int32)
    pad = e_pad - n_edges
    src3 = jnp.concatenate([src, jnp.zeros((pad,), jnp.int32)]).reshape(
        _NW, chunks, _CHUNK)
    dst3 = jnp.concatenate([dst, jnp.full((pad,), _DUMMY, jnp.int32)]).reshape(
        _NW, chunks, _CHUNK)
    zeros_tile = jnp.zeros((_TILE_ROWS, _DO), jnp.float32)

    bm = b_mlp.reshape(1, _DO)
    bc = b_conv.reshape(1, _DO)
    bi = b_ih.reshape(1, 3 * _DO)
    bh = b_hh.reshape(1, 3 * _DO)

    sc_scatter = _make_sc_scatter(chunks)

    h, lin = _tc_mlp_conv(x, W_mlp, bm, W_conv, bc)
    for _ in range(_STEPS):
        p0, p1 = sc_scatter(lin, src3, dst3, zeros_tile)
        h, lin = _tc_gru(p0, p1, x, h, W_ih, W_hh, bi, bh, W_conv, bc)
    return h


# speedup vs baseline: 9.9367x; 1.0043x over previous
import functools

import jax
import jax.numpy as jnp
from jax import lax
from jax.experimental import pallas as pl
from jax.experimental.pallas import tpu as pltpu
from jax.experimental.pallas import tpu_sc as plsc

_N = 10000
_DI = 128
_DO = 64
_STEPS = 2

_NC = 2
_NS = 16
_NW = _NC * _NS
_CHUNK = 128
_TILE_ROWS = 640
_ACC_ROWS = _NS * _TILE_ROWS
_DUMMY = _N


def _mlp_conv_body(x_ref, wm_ref, bm_ref, wc_ref, bc_ref, h_ref, lin_ref):
    o = jnp.dot(x_ref[...], wm_ref[...], preferred_element_type=jnp.float32)
    o = o + bm_ref[...]
    h_ref[...] = o
    lin = jnp.dot(o, wc_ref[...], preferred_element_type=jnp.float32)
    lin_ref[...] = lin + bc_ref[...]


def _dot_t(a, w):
    return lax.dot_general(a, w, (((1,), (1,)), ((), ())),
                           preferred_element_type=jnp.float32)


def _gru_body(p0_ref, p1_ref, x_ref, h_ref, wih_ref, whh_ref,
              bi_ref, bh_ref, wc_ref, bc_ref, hn_ref, lin_ref):
    m = p0_ref[...] + p1_ref[...]
    wih = wih_ref[...]
    gi = (
        _dot_t(m, wih[:, :_DO])
        + _dot_t(x_ref[...], wih[:, _DO:])
        + bi_ref[...]
    )
    gh = _dot_t(h_ref[...], whh_ref[...])
    gh = gh + bh_ref[...]
    h = h_ref[...]
    r = jax.nn.sigmoid(gi[:, 0:_DO] + gh[:, 0:_DO])
    z = jax.nn.sigmoid(gi[:, _DO:2 * _DO] + gh[:, _DO:2 * _DO])
    n = jnp.tanh(gi[:, 2 * _DO:3 * _DO] + r * gh[:, 2 * _DO:3 * _DO])
    hn = (1.0 - z) * n + z * h
    hn_ref[...] = hn
    if lin_ref is not None:
        lin_ref[...] = (
            jnp.dot(hn, wc_ref[...], preferred_element_type=jnp.float32)
            + bc_ref[...]
        )


_G = 5
_BLK = _N // _G


def _tc_mlp_conv(x, W_mlp, b_mlp, W_conv, b_conv):
    return pl.pallas_call(
        _mlp_conv_body,
        grid=(_G,),
        in_specs=[
            pl.BlockSpec((_BLK, _DI), lambda i: (i, 0)),
            pl.BlockSpec((_DI, _DO), lambda i: (0, 0)),
            pl.BlockSpec((1, _DO), lambda i: (0, 0)),
            pl.BlockSpec((_DO, _DO), lambda i: (0, 0)),
            pl.BlockSpec((1, _DO), lambda i: (0, 0)),
        ],
        out_specs=[
            pl.BlockSpec((_BLK, _DO), lambda i: (i, 0)),
            pl.BlockSpec((_BLK, _DO), lambda i: (i, 0)),
        ],
        out_shape=[
            jax.ShapeDtypeStruct((_N, _DO), jnp.float32),
            jax.ShapeDtypeStruct((_ACC_ROWS, _DO), jnp.float32),
        ],
    )(x, W_mlp, b_mlp, W_conv, b_conv)


def _tc_gru(p0, p1, x, h, wih, whh, bi, bh, wc, bc, want_lin):
    body = _gru_body if want_lin else (
        lambda *refs: _gru_body(*refs, None))
    out_specs = [pl.BlockSpec((_BLK, _DO), lambda i: (i, 0))]
    out_shape = [jax.ShapeDtypeStruct((_N, _DO), jnp.float32)]
    if want_lin:
        out_specs.append(pl.BlockSpec((_BLK, _DO), lambda i: (i, 0)))
        out_shape.append(
            jax.ShapeDtypeStruct((_ACC_ROWS, _DO), jnp.float32))
    return pl.pallas_call(
        body,
        grid=(_G,),
        in_specs=[
            pl.BlockSpec((_BLK, _DO), lambda i: (i, 0)),
            pl.BlockSpec((_BLK, _DO), lambda i: (i, 0)),
            pl.BlockSpec((_BLK, _DI), lambda i: (i, 0)),
            pl.BlockSpec((_BLK, _DO), lambda i: (i, 0)),
            pl.BlockSpec((3 * _DO, _DO + _DI), lambda i: (0, 0)),
            pl.BlockSpec((3 * _DO, _DO), lambda i: (0, 0)),
            pl.BlockSpec((1, 3 * _DO), lambda i: (0, 0)),
            pl.BlockSpec((1, 3 * _DO), lambda i: (0, 0)),
            pl.BlockSpec((_DO, _DO), lambda i: (0, 0)),
            pl.BlockSpec((1, _DO), lambda i: (0, 0)),
        ],
        out_specs=out_specs,
        out_shape=out_shape,
    )(p0, p1, x, h, wih, whh, bi, bh, wc, bc)


_NBUF = 4


def _make_sc_scatter(chunks):
    mesh = plsc.VectorSubcoreMesh(core_axis_name="c", subcore_axis_name="s")
    groups = chunks // _NBUF

    @functools.partial(
        pl.kernel,
        mesh=mesh,
        out_type=[
            jax.ShapeDtypeStruct((_ACC_ROWS, _DO), jnp.float32),
            jax.ShapeDtypeStruct((_ACC_ROWS, _DO), jnp.float32),
        ],
        compiler_params=pltpu.CompilerParams(use_tc_tiling_on_sc=False),
        scratch_types=[
            pltpu.VMEM((chunks, _CHUNK), jnp.int32),
            pltpu.VMEM((chunks, _CHUNK), jnp.int32),
            pltpu.VMEM((_CHUNK, _DO), jnp.float32),
            pltpu.VMEM_SHARED((_ACC_ROWS, _DO), jnp.float32),
            pltpu.VMEM_SHARED((_ACC_ROWS, _DO), jnp.float32),
            pltpu.SemaphoreType.DMA,
        ],
    )
    def sc_scatter(lin_hbm, src_hbm, dst_hbm, zeros_hbm, out0_hbm, out1_hbm,
                   src_v, dst_v, rows_v, lin_sh, acc_sh, sem):
        c = lax.axis_index("c")
        s = lax.axis_index("s")
        wid = s * _NC + c
        row0 = s * _TILE_ROWS
        stripe = pl.ds(row0, _TILE_ROWS)
        pltpu.sync_copy(lin_hbm.at[stripe], lin_sh.at[stripe])
        pltpu.sync_copy(zeros_hbm, acc_sh.at[stripe])
        pltpu.sync_copy(src_hbm.at[wid], src_v)
        pltpu.sync_copy(dst_hbm.at[wid], dst_v)
        plsc.subcore_barrier()

        def chunk(j, carry):
            pltpu.async_copy(lin_sh.at[src_v.at[j]], rows_v, sem).wait()
            pltpu.sync_copy(rows_v, acc_sh.at[dst_v.at[j]], add=True)
            return carry

        lax.fori_loop(0, chunks, chunk, 0)
        plsc.subcore_barrier()

        @pl.when(c == 0)
        def _():
            pltpu.sync_copy(acc_sh.at[stripe], out0_hbm.at[stripe])

        @pl.when(c == 1)
        def _():
            pltpu.sync_copy(acc_sh.at[stripe], out1_hbm.at[stripe])

    return sc_scatter


def kernel(x, edge_index, W_mlp, b_mlp, W_conv, b_conv, W_ih, W_hh, b_ih, b_hh):
    n_edges = edge_index.shape[1]
    e_per_w = -(-n_edges // (_NW * _CHUNK)) * _CHUNK
    chunks = e_per_w // _CHUNK
    e_pad = _NW * e_per_w

    src = edge_index[0].astype(jnp.int32)
    dst = edge_index[1].astype(jnp.int32)
    pad = e_pad - n_edges
    src3 = jnp.concatenate([src, jnp.zeros((pad,), jnp.int32)]).reshape(
        _NW, chunks, _CHUNK)
    dst3 = jnp.concatenate([dst, jnp.full((pad,), _DUMMY, jnp.int32)]).reshape(
        _NW, chunks, _CHUNK)
    zeros_tile = jnp.zeros((_TILE_ROWS, _DO), jnp.float32)

    bm = b_mlp.reshape(1, _DO)
    bc = b_conv.reshape(1, _DO)
    bi = b_ih.reshape(1, 3 * _DO)
    bh = b_hh.reshape(1, 3 * _DO)

    sc_scatter = _make_sc_scatter(chunks)

    h, lin = _tc_mlp_conv(x, W_mlp, bm, W_conv, bc)
    for step in range(_STEPS):
        p0, p1 = sc_scatter(lin, src3, dst3, zeros_tile)
        want_lin = step < _STEPS - 1
        outs = _tc_gru(p0, p1, x, h, W_ih, W_hh, bi, bh, W_conv, bc, want_lin)
        if want_lin:
            h, lin = outs
        else:
            (h,) = outs
    return h


# speedup vs baseline: 9.9465x; 1.0010x over previous
import functools

import jax
import jax.numpy as jnp
from jax import lax
from jax.experimental import pallas as pl
from jax.experimental.pallas import tpu as pltpu
from jax.experimental.pallas import tpu_sc as plsc

_N = 10000
_DI = 128
_DO = 64
_STEPS = 2

_NC = 2
_NS = 16
_NW = _NC * _NS
_CHUNK = 128
_TILE_ROWS = 640
_ACC_ROWS = _NS * _TILE_ROWS
_DUMMY = _N


def _mlp_conv_body(x_ref, wm_ref, bm_ref, wc_ref, bc_ref, h_ref, lin_ref):
    o = jnp.dot(x_ref[...], wm_ref[...], preferred_element_type=jnp.float32)
    o = o + bm_ref[...]
    h_ref[...] = o
    lin = jnp.dot(o, wc_ref[...], preferred_element_type=jnp.float32)
    lin_ref[...] = lin + bc_ref[...]


def _dot_t(a, w):
    return lax.dot_general(a, w, (((1,), (1,)), ((), ())),
                           preferred_element_type=jnp.float32)


def _gru_body(p0_ref, p1_ref, x_ref, h_ref, wih_ref, whh_ref,
              bi_ref, bh_ref, wc_ref, bc_ref, hn_ref, lin_ref):
    m = p0_ref[...] + p1_ref[...]
    wih = wih_ref[...]
    gi = (
        _dot_t(m, wih[:, :_DO])
        + _dot_t(x_ref[...], wih[:, _DO:])
        + bi_ref[...]
    )
    gh = _dot_t(h_ref[...], whh_ref[...])
    gh = gh + bh_ref[...]
    h = h_ref[...]
    r = jax.nn.sigmoid(gi[:, 0:_DO] + gh[:, 0:_DO])
    z = jax.nn.sigmoid(gi[:, _DO:2 * _DO] + gh[:, _DO:2 * _DO])
    n = jnp.tanh(gi[:, 2 * _DO:3 * _DO] + r * gh[:, 2 * _DO:3 * _DO])
    hn = (1.0 - z) * n + z * h
    hn_ref[...] = hn
    if lin_ref is not None:
        lin_ref[...] = (
            jnp.dot(hn, wc_ref[...], preferred_element_type=jnp.float32)
            + bc_ref[...]
        )


_G = 5
_BLK = _N // _G


def _tc_mlp_conv(x, W_mlp, b_mlp, W_conv, b_conv):
    return pl.pallas_call(
        _mlp_conv_body,
        grid=(_G,),
        in_specs=[
            pl.BlockSpec((_BLK, _DI), lambda i: (i, 0)),
            pl.BlockSpec((_DI, _DO), lambda i: (0, 0)),
            pl.BlockSpec((1, _DO), lambda i: (0, 0)),
            pl.BlockSpec((_DO, _DO), lambda i: (0, 0)),
            pl.BlockSpec((1, _DO), lambda i: (0, 0)),
        ],
        out_specs=[
            pl.BlockSpec((_BLK, _DO), lambda i: (i, 0)),
            pl.BlockSpec((_BLK, _DO), lambda i: (i, 0)),
        ],
        out_shape=[
            jax.ShapeDtypeStruct((_N, _DO), jnp.float32),
            jax.ShapeDtypeStruct((_ACC_ROWS, _DO), jnp.float32),
        ],
    )(x, W_mlp, b_mlp, W_conv, b_conv)


def _tc_gru(p0, p1, x, h, wih, whh, bi, bh, wc, bc, want_lin):
    body = _gru_body if want_lin else (
        lambda *refs: _gru_body(*refs, None))
    out_specs = [pl.BlockSpec((_BLK, _DO), lambda i: (i, 0))]
    out_shape = [jax.ShapeDtypeStruct((_N, _DO), jnp.float32)]
    if want_lin:
        out_specs.append(pl.BlockSpec((_BLK, _DO), lambda i: (i, 0)))
        out_shape.append(
            jax.ShapeDtypeStruct((_ACC_ROWS, _DO), jnp.float32))
    return pl.pallas_call(
        body,
        grid=(_G,),
        in_specs=[
            pl.BlockSpec((_BLK, _DO), lambda i: (i, 0)),
            pl.BlockSpec((_BLK, _DO), lambda i: (i, 0)),
            pl.BlockSpec((_BLK, _DI), lambda i: (i, 0)),
            pl.BlockSpec((_BLK, _DO), lambda i: (i, 0)),
            pl.BlockSpec((3 * _DO, _DO + _DI), lambda i: (0, 0)),
            pl.BlockSpec((3 * _DO, _DO), lambda i: (0, 0)),
            pl.BlockSpec((1, 3 * _DO), lambda i: (0, 0)),
            pl.BlockSpec((1, 3 * _DO), lambda i: (0, 0)),
            pl.BlockSpec((_DO, _DO), lambda i: (0, 0)),
            pl.BlockSpec((1, _DO), lambda i: (0, 0)),
        ],
        out_specs=out_specs,
        out_shape=out_shape,
    )(p0, p1, x, h, wih, whh, bi, bh, wc, bc)


def _make_sc_scatter(chunks):
    mesh = plsc.VectorSubcoreMesh(core_axis_name="c", subcore_axis_name="s")

    @functools.partial(
        pl.kernel,
        mesh=mesh,
        out_type=[
            jax.ShapeDtypeStruct((_ACC_ROWS, _DO), jnp.float32),
            jax.ShapeDtypeStruct((_ACC_ROWS, _DO), jnp.float32),
        ],
        compiler_params=pltpu.CompilerParams(use_tc_tiling_on_sc=False),
        scratch_types=[
            pltpu.VMEM((chunks, _CHUNK), jnp.int32),
            pltpu.VMEM((chunks, _CHUNK), jnp.int32),
            pltpu.VMEM((_CHUNK, _DO), jnp.float32),
            pltpu.VMEM_SHARED((_ACC_ROWS, _DO), jnp.float32),
            pltpu.VMEM_SHARED((_ACC_ROWS, _DO), jnp.float32),
            pltpu.SemaphoreType.DMA,
        ],
    )
    def sc_scatter(lin_hbm, src_hbm, dst_hbm, zeros_hbm, out0_hbm, out1_hbm,
                   src_v, dst_v, rows_v, lin_sh, acc_sh, sem):
        c = lax.axis_index("c")
        s = lax.axis_index("s")
        wid = s * _NC + c
        row0 = s * _TILE_ROWS
        stripe = pl.ds(row0, _TILE_ROWS)
        pltpu.sync_copy(lin_hbm.at[stripe], lin_sh.at[stripe])
        pltpu.sync_copy(zeros_hbm, acc_sh.at[stripe])
        pltpu.sync_copy(src_hbm.at[wid], src_v)
        pltpu.sync_copy(dst_hbm.at[wid], dst_v)
        plsc.subcore_barrier()

        def chunk(j, carry):
            pltpu.async_copy(lin_sh.at[src_v.at[j]], rows_v, sem).wait()
            pltpu.sync_copy(rows_v, acc_sh.at[dst_v.at[j]], add=True)
            return carry

        lax.fori_loop(0, chunks, chunk, 0)
        plsc.subcore_barrier()

        @pl.when(c == 0)
        def _():
            pltpu.sync_copy(acc_sh.at[stripe], out0_hbm.at[stripe])

        @pl.when(c == 1)
        def _():
            pltpu.sync_copy(acc_sh.at[stripe], out1_hbm.at[stripe])

    return sc_scatter


def kernel(x, edge_index, W_mlp, b_mlp, W_conv, b_conv, W_ih, W_hh, b_ih, b_hh):
    n_edges = edge_index.shape[1]
    e_per_w = -(-n_edges // (_NW * _CHUNK)) * _CHUNK
    chunks = e_per_w // _CHUNK
    e_pad = _NW * e_per_w

    src = edge_index[0].astype(jnp.int32)
    dst = edge_index[1].astype(jnp.int32)
    pad = e_pad - n_edges
    src3 = jnp.concatenate([src, jnp.zeros((pad,), jnp.int32)]).reshape(
        _NW, chunks, _CHUNK)
    dst3 = jnp.concatenate([dst, jnp.full((pad,), _DUMMY, jnp.int32)]).reshape(
        _NW, chunks, _CHUNK)
    zeros_tile = jnp.zeros((_TILE_ROWS, _DO), jnp.float32)

    bm = b_mlp.reshape(1, _DO)
    bc = b_conv.reshape(1, _DO)
    bi = b_ih.reshape(1, 3 * _DO)
    bh = b_hh.reshape(1, 3 * _DO)

    sc_scatter = _make_sc_scatter(chunks)

    h, lin = _tc_mlp_conv(x, W_mlp, bm, W_conv, bc)
    for step in range(_STEPS):
        p0, p1 = sc_scatter(lin, src3, dst3, zeros_tile)
        want_lin = step < _STEPS - 1
        outs = _tc_gru(p0, p1, x, h, W_ih, W_hh, bi, bh, W_conv, bc, want_lin)
        if want_lin:
            h, lin = outs
        else:
            (h,) = outs
    return h
